# baseline (speedup 1.0000x reference)
import jax
import jax.numpy as jnp
from jax import lax
from jax.experimental import pallas as pl
from jax.experimental.pallas import tpu as pltpu
from jax.experimental.pallas import tpu_sc as plsc

_B, _L, _DY, _DX, _DM = 8, 512, 32, 6, 128
_K = 6
_NC, _NS = 2, 16


_DC = 16

_S0 = 0.99999986216691
_S1 = -0.16666607728014005
_S2 = 0.008332732437814282
_S3 = -0.0001981669232761085
_S4 = 2.708326132222227e-06
_S5 = -2.069597015432612e-08
_INV_2PI = 0.15915494309189535
_2PI_HI = 6.28125
_2PI_LO = 1.9353071795864769e-03


def _fast_sin(a):
    k = jnp.round(a * _INV_2PI)
    r = a - k * _2PI_HI - k * _2PI_LO
    r2 = r * r
    return r * (_S0 + r2 * (_S1 + r2 * (_S2 + r2 * (
        _S3 + r2 * (_S4 + r2 * _S5)))))


def _sc_body(space_hbm, out_hbm, idx_v, buf, sem):
    wid = lax.axis_index("s") * _NC + lax.axis_index("c")
    val = jnp.full((16,), wid, dtype=jnp.int32)
    for j in range(8):
        idx_v[pl.ds(j * 16, 16)] = val
    gathers = [pltpu.async_copy(
        space_hbm.at[idx_v], buf.at[pl.ds(q * 128, 128)], sem)
        for q in range(4)]
    for g in gathers:
        g.wait()
    copies = [pltpu.async_copy(
        buf, out_hbm.at[pl.ds(b * _DY * _L + wid * _L, _L)], sem)
        for b in range(_B)]
    for cp in copies:
        cp.wait()


def _sc_space_emb(space_table):
    run = pl.kernel(
        _sc_body,
        out_type=jax.ShapeDtypeStruct((_B * _DY * _L, _DM), jnp.float32),
        mesh=plsc.VectorSubcoreMesh(core_axis_name="c", subcore_axis_name="s"),
        scratch_types=[
            pltpu.VMEM((128,), jnp.int32),
            pltpu.VMEM((_L, _DM), jnp.float32),
            pltpu.SemaphoreType.DMA,
        ],
    )
    return run(space_table)


def _tc_body(x_ref, y4_ref, t2vw_ref, t2vb_ref, local_ref, vtw_ref, vtb_ref,
             given_ref, out1_ref, out3_ref, base_ref):
    c = pl.program_id(1)

    @pl.when(c == 0)
    def _compute_base():
        xs = x_ref[0]
        xs = jnp.where(jnp.isnan(xs), 0.0, xs)
        aff = xs * t2vw_ref[...] + t2vb_ref[...]
        col = lax.broadcasted_iota(jnp.int32, aff.shape, 1)
        te = jnp.where(col % _K == 0, aff, _fast_sin(aff))
        base_ref[...] = (local_ref[...] + vtb_ref[...] + given_ref[1:2, :]
                         + jnp.dot(te, vtw_ref[0:_DX * _K, :],
                                   preferred_element_type=jnp.float32))

    w_y = vtw_ref[_DX * _K:_DX * _K + 1, :]
    delta = given_ref[0:1, :] - given_ref[1:2, :]
    base = base_ref[...]
    for j in range(_DC):
        yc = y4_ref[0, j]
        nan = jnp.isnan(yc)
        ycc = jnp.where(nan, 0.0, yc)
        out1_ref[0, j * _L:(j + 1) * _L, :] = (
            base + ycc * w_y + jnp.where(nan, 1.0, 0.0) * delta)
        out3_ref[j] = jnp.full((1, _L), c * _DC + j, dtype=jnp.int32)


def kernel(x, y, t2v_w, t2v_b, local_table, vt_w, vt_b, space_table,
           given_table):
    batch, length, dy = y.shape
    x36 = jnp.repeat(x.reshape(batch, length, _DX), _K, axis=-1)
    wflat = t2v_w.reshape(1, _DX * _K)
    bflat = t2v_b.reshape(1, _DX * _K)
    y4 = jnp.transpose(y, (0, 2, 1)).reshape(batch, dy, length, 1)
    vtb2 = vt_b.reshape(1, _DM)

    out2 = _sc_space_emb(space_table).reshape(batch, dy * length, _DM)

    nc = dy // _DC
    grid = (batch, nc)
    out1, out3 = pl.pallas_call(
        _tc_body,
        grid=grid,
        in_specs=[
            pl.BlockSpec((1, length, _DX * _K), lambda b, c: (b, 0, 0)),
            pl.BlockSpec((1, _DC, length, 1), lambda b, c: (b, c, 0, 0)),
            pl.BlockSpec((1, _DX * _K), lambda b, c: (0, 0)),
            pl.BlockSpec((1, _DX * _K), lambda b, c: (0, 0)),
            pl.BlockSpec((length, _DM), lambda b, c: (0, 0)),
            pl.BlockSpec((_DX * _K + 1, _DM), lambda b, c: (0, 0)),
            pl.BlockSpec((1, _DM), lambda b, c: (0, 0)),
            pl.BlockSpec((2, _DM), lambda b, c: (0, 0)),
        ],
        out_specs=[
            pl.BlockSpec((1, _DC * length, _DM), lambda b, c: (b, c, 0)),
            pl.BlockSpec((_DC, 1, length), lambda b, c: (b * nc + c, 0, 0)),
        ],
        out_shape=[
            jax.ShapeDtypeStruct((batch, dy * length, _DM), jnp.float32),
            jax.ShapeDtypeStruct((batch * dy, 1, length), jnp.int32),
        ],
        scratch_shapes=[pltpu.VMEM((length, _DM), jnp.float32)],
        compiler_params=pltpu.CompilerParams(
            dimension_semantics=("arbitrary", "arbitrary")),
    )(x36, y4, wflat, bflat, local_table[:length], vt_w, vtb2,
      given_table)

    return (out1, out2, out3.reshape(batch, dy * length))


# speedup vs baseline: 2.2987x; 2.2987x over previous
import jax
import jax.numpy as jnp
from jax import lax
from jax.experimental import pallas as pl
from jax.experimental.pallas import tpu as pltpu

_B, _L, _DY, _DX, _DM = 8, 512, 32, 6, 128
_K = 6

_S0 = 0.99999986216691
_S1 = -0.16666607728014005
_S2 = 0.008332732437814282
_S3 = -0.0001981669232761085
_S4 = 2.708326132222227e-06
_S5 = -2.069597015432612e-08
_INV_2PI = 0.15915494309189535
_2PI_HI = 6.28125
_2PI_LO = 1.9353071795864769e-03


def _fast_sin(a):
    k = jnp.round(a * _INV_2PI)
    r = a - k * _2PI_HI - k * _2PI_LO
    r2 = r * r
    return r * (_S0 + r2 * (_S1 + r2 * (_S2 + r2 * (
        _S3 + r2 * (_S4 + r2 * _S5)))))


def _tc_body(x_ref, y_ref, t2vw_ref, t2vb_ref, local_ref, vtw_ref, vtb_ref,
             space_ref, given_ref, out1_ref, out2_ref, out3_ref, base_ref):
    xs = x_ref[0]
    xs = jnp.where(jnp.isnan(xs), 0.0, xs)
    aff = xs * t2vw_ref[...] + t2vb_ref[...]
    col = lax.broadcasted_iota(jnp.int32, aff.shape, 1)
    te = jnp.where(col % _K == 0, aff, _fast_sin(aff))
    base_ref[...] = (local_ref[...] + vtb_ref[...] + given_ref[1:2, :]
                     + jnp.dot(te, vtw_ref[0:_DX * _K, :],
                               preferred_element_type=jnp.float32))

    w_y = vtw_ref[_DX * _K:_DX * _K + 1, :]
    delta = given_ref[0:1, :] - given_ref[1:2, :]
    ys = y_ref[0]
    base = base_ref[...]
    for j in range(_DY):
        yc = ys[:, j:j + 1]
        nan = jnp.isnan(yc)
        ycc = jnp.where(nan, 0.0, yc)
        out1_ref[0, j * _L:(j + 1) * _L, :] = (
            base + ycc * w_y + jnp.where(nan, 1.0, 0.0) * delta)
        out2_ref[0, j * _L:(j + 1) * _L, :] = jnp.broadcast_to(
            space_ref[j:j + 1, :], (_L, _DM))
        out3_ref[0, 0:1, j * _L:(j + 1) * _L] = jnp.full(
            (1, _L), j, dtype=jnp.int32)


def kernel(x, y, t2v_w, t2v_b, local_table, vt_w, vt_b, space_table,
           given_table):
    batch, length, dy = y.shape
    x36 = jnp.repeat(x.reshape(batch, length, _DX), _K, axis=-1)
    wflat = t2v_w.reshape(1, _DX * _K)
    bflat = t2v_b.reshape(1, _DX * _K)
    vtb2 = vt_b.reshape(1, _DM)

    out1, out2, out3 = pl.pallas_call(
        _tc_body,
        grid=(batch,),
        in_specs=[
            pl.BlockSpec((1, length, _DX * _K), lambda b: (b, 0, 0)),
            pl.BlockSpec((1, length, dy), lambda b: (b, 0, 0)),
            pl.BlockSpec((1, _DX * _K), lambda b: (0, 0)),
            pl.BlockSpec((1, _DX * _K), lambda b: (0, 0)),
            pl.BlockSpec((length, _DM), lambda b: (0, 0)),
            pl.BlockSpec((_DX * _K + 1, _DM), lambda b: (0, 0)),
            pl.BlockSpec((1, _DM), lambda b: (0, 0)),
            pl.BlockSpec((_DY, _DM), lambda b: (0, 0)),
            pl.BlockSpec((2, _DM), lambda b: (0, 0)),
        ],
        out_specs=[
            pl.BlockSpec((1, dy * length, _DM), lambda b: (b, 0, 0)),
            pl.BlockSpec((1, dy * length, _DM), lambda b: (b, 0, 0)),
            pl.BlockSpec((1, 1, dy * length), lambda b: (b, 0, 0)),
        ],
        out_shape=[
            jax.ShapeDtypeStruct((batch, dy * length, _DM), jnp.float32),
            jax.ShapeDtypeStruct((batch, dy * length, _DM), jnp.float32),
            jax.ShapeDtypeStruct((batch, 1, dy * length), jnp.int32),
        ],
        scratch_shapes=[pltpu.VMEM((length, _DM), jnp.float32)],
        compiler_params=pltpu.CompilerParams(
            dimension_semantics=("arbitrary",)),
    )(x36, y, wflat, bflat, local_table[:length], vt_w, vtb2, space_table,
      given_table)

    return (out1, out2, out3.reshape(batch, dy * length))


# speedup vs baseline: 2.3072x; 1.0037x over previous
import jax
import jax.numpy as jnp
from jax import lax
from jax.experimental import pallas as pl
from jax.experimental.pallas import tpu as pltpu

_B, _L, _DY, _DX, _DM = 8, 512, 32, 6, 128
_K = 6

_S0 = 0.99999986216691
_S1 = -0.16666607728014005
_S2 = 0.008332732437814282
_S3 = -0.0001981669232761085
_S4 = 2.708326132222227e-06
_S5 = -2.069597015432612e-08
_INV_2PI = 0.15915494309189535
_2PI_HI = 6.28125
_2PI_LO = 1.9353071795864769e-03


def _fast_sin(a):
    k = jnp.round(a * _INV_2PI)
    r = a - k * _2PI_HI - k * _2PI_LO
    r2 = r * r
    return r * (_S0 + r2 * (_S1 + r2 * (_S2 + r2 * (
        _S3 + r2 * (_S4 + r2 * _S5)))))


def _tc_body(x_ref, y_ref, t2vw_ref, t2vb_ref, local_ref, vtw_ref, vtb_ref,
             space_ref, given_ref, out1_ref, out2_ref, out3_ref, base_ref):
    xs = x_ref[0]
    xs = jnp.where(jnp.isnan(xs), 0.0, xs)
    aff = xs * t2vw_ref[...] + t2vb_ref[...]
    col = lax.broadcasted_iota(jnp.int32, aff.shape, 1)
    te = jnp.where(col % _K == 0, aff, _fast_sin(aff))
    base_ref[...] = (local_ref[...] + vtb_ref[...] + given_ref[1:2, :]
                     + jnp.dot(te, vtw_ref[0:_DX * _K, :],
                               preferred_element_type=jnp.float32))

    w_y = vtw_ref[_DX * _K:_DX * _K + 1, :]
    delta = given_ref[0:1, :] - given_ref[1:2, :]
    wd = jnp.concatenate([w_y, delta], axis=0)
    ys = y_ref[0]
    base = base_ref[...]
    for j in range(_DY):
        yc = ys[:, j:j + 1]
        nan = jnp.isnan(yc)
        a = jnp.concatenate(
            [jnp.where(nan, 0.0, yc), jnp.where(nan, 1.0, 0.0)], axis=1)
        out1_ref[0, j * _L:(j + 1) * _L, :] = base + jnp.dot(
            a, wd, preferred_element_type=jnp.float32)
        out2_ref[0, j * _L:(j + 1) * _L, :] = jnp.broadcast_to(
            space_ref[j:j + 1, :], (_L, _DM))
        out3_ref[0, 0:1, j * _L:(j + 1) * _L] = jnp.full(
            (1, _L), j, dtype=jnp.int32)


def kernel(x, y, t2v_w, t2v_b, local_table, vt_w, vt_b, space_table,
           given_table):
    batch, length, dy = y.shape
    x36 = jnp.repeat(x.reshape(batch, length, _DX), _K, axis=-1)
    wflat = t2v_w.reshape(1, _DX * _K)
    bflat = t2v_b.reshape(1, _DX * _K)
    vtb2 = vt_b.reshape(1, _DM)

    out1, out2, out3 = pl.pallas_call(
        _tc_body,
        grid=(batch,),
        in_specs=[
            pl.BlockSpec((1, length, _DX * _K), lambda b: (b, 0, 0)),
            pl.BlockSpec((1, length, dy), lambda b: (b, 0, 0)),
            pl.BlockSpec((1, _DX * _K), lambda b: (0, 0)),
            pl.BlockSpec((1, _DX * _K), lambda b: (0, 0)),
            pl.BlockSpec((length, _DM), lambda b: (0, 0)),
            pl.BlockSpec((_DX * _K + 1, _DM), lambda b: (0, 0)),
            pl.BlockSpec((1, _DM), lambda b: (0, 0)),
            pl.BlockSpec((_DY, _DM), lambda b: (0, 0)),
            pl.BlockSpec((2, _DM), lambda b: (0, 0)),
        ],
        out_specs=[
            pl.BlockSpec((1, dy * length, _DM), lambda b: (b, 0, 0)),
            pl.BlockSpec((1, dy * length, _DM), lambda b: (b, 0, 0)),
            pl.BlockSpec((1, 1, dy * length), lambda b: (b, 0, 0)),
        ],
        out_shape=[
            jax.ShapeDtypeStruct((batch, dy * length, _DM), jnp.float32),
            jax.ShapeDtypeStruct((batch, dy * length, _DM), jnp.float32),
            jax.ShapeDtypeStruct((batch, 1, dy * length), jnp.int32),
        ],
        scratch_shapes=[pltpu.VMEM((length, _DM), jnp.float32)],
        compiler_params=pltpu.CompilerParams(
            dimension_semantics=("arbitrary",)),
    )(x36, y, wflat, bflat, local_table[:length], vt_w, vtb2, space_table,
      given_table)

    return (out1, out2, out3.reshape(batch, dy * length))
